# Initial kernel scaffold; baseline (speedup 1.0000x reference)
import jax
import jax.numpy as jnp
from jax import lax
from jax.experimental import pallas as pl
from jax.experimental.pallas import tpu as pltpu
from jax.experimental.pallas import tpu_sc as plsc

N = 10000
E = 320000
D = 128
NC = 2
NS = 16
NW = NC * NS
L = 16
EW = E // NW
KB = 80
NBLK = EW // KB
NP = 10240
DSL = NP // NS
ORT = N // NS
ZR = 25

_MESH = plsc.VectorSubcoreMesh(core_axis_name="c", subcore_axis_name="s")


def _tc_head1_body(x_ref, w_ref, asr_ref, adr_ref, h_ref, as_ref, ad_ref):
    h = jnp.dot(x_ref[...], w_ref[...], preferred_element_type=jnp.float32)
    h_ref[...] = h
    as_ref[...] = jnp.sum(h * asr_ref[...][None, :], axis=1)
    ad_ref[...] = jnp.sum(h * adr_ref[...][None, :], axis=1)


def _tc_head2_body(p_ref, b_ref, w_ref, asr_ref, adr_ref, h_ref, as_ref,
                   ad_ref):
    hid = jnp.maximum(p_ref[0] + p_ref[1] + b_ref[...][None, :], 0.0)
    h = jnp.dot(hid, w_ref[...], preferred_element_type=jnp.float32)
    h_ref[...] = h
    as_ref[...] = jnp.sum(h * asr_ref[...][None, :], axis=1)
    ad_ref[...] = jnp.sum(h * adr_ref[...][None, :], axis=1)


def _tc_out_body(p_ref, b_ref, o_ref):
    o = p_ref[0] + p_ref[1] + b_ref[...][None, :]
    m = jnp.max(o, axis=1, keepdims=True)
    ex = jnp.exp(o - m)
    o_ref[...] = o - m - jnp.log(jnp.sum(ex, axis=1, keepdims=True))


_BN = 2000

_head_out = [
    jax.ShapeDtypeStruct((N, D), jnp.float32),
    jax.ShapeDtypeStruct((N,), jnp.float32),
    jax.ShapeDtypeStruct((N,), jnp.float32),
]
_head_out_specs = [
    pl.BlockSpec((_BN, D), lambda i: (i, 0)),
    pl.BlockSpec((_BN,), lambda i: (i,)),
    pl.BlockSpec((_BN,), lambda i: (i,)),
]
_full_mat = pl.BlockSpec((D, D), lambda i: (0, 0))
_full_vec = pl.BlockSpec((D,), lambda i: (0,))


def _tc_head1(x, W, a_src, a_dst):
    return pl.pallas_call(
        _tc_head1_body,
        grid=(N // _BN,),
        in_specs=[pl.BlockSpec((_BN, D), lambda i: (i, 0)), _full_mat,
                  _full_vec, _full_vec],
        out_specs=_head_out_specs,
        out_shape=_head_out,
    )(x, W, a_src, a_dst)


def _tc_head2(p, b, W, a_src, a_dst):
    return pl.pallas_call(
        _tc_head2_body,
        grid=(N // _BN,),
        in_specs=[pl.BlockSpec((NC, _BN, D), lambda i: (0, i, 0)), _full_vec,
                  _full_mat, _full_vec, _full_vec],
        out_specs=_head_out_specs,
        out_shape=_head_out,
    )(p, b, W, a_src, a_dst)


def _tc_out(p, b):
    return pl.pallas_call(
        _tc_out_body,
        grid=(N // _BN,),
        in_specs=[pl.BlockSpec((NC, _BN, D), lambda i: (0, i, 0)), _full_vec],
        out_specs=pl.BlockSpec((_BN, D), lambda i: (i, 0)),
        out_shape=jax.ShapeDtypeStruct((N, D), jnp.float32),
    )(p, b)


def _sc_edge_body(src_hbm, dst_hbm, as_hbm, ad_hbm, eexp_hbm, dpart_hbm,
                  src_v, dst_v, as_v, ad_v, eexp_v, denom_v, red_a, red_b,
                  dsh):
    cid = lax.axis_index("c")
    sid = lax.axis_index("s")
    w = cid * NS + sid

    pltpu.sync_copy(src_hbm.at[w], src_v)
    pltpu.sync_copy(dst_hbm.at[w], dst_v)
    pltpu.sync_copy(as_hbm, as_v)
    pltpu.sync_copy(ad_hbm, ad_v)

    zv = jnp.zeros((L,), jnp.float32)

    def zero_body(i, c):
        denom_v[pl.ds(i * L, L)] = zv
        return c
    lax.fori_loop(0, NP // L, zero_body, 0)

    neg = jnp.full((L,), -1e30, jnp.float32)

    def max_body(i, carry):
        ms, md = carry
        ms = jnp.maximum(ms, as_v[pl.ds(i * L, L)])
        md = jnp.maximum(md, ad_v[pl.ds(i * L, L)])
        return ms, md
    ms, md = lax.fori_loop(0, N // L, max_body, (neg, neg))
    m_tot = jnp.max(ms) + jnp.max(md)
    m_tot = jnp.where(m_tot >= 0.0, m_tot, 0.2 * m_tot)

    def edge_body(i, c):
        s16 = src_v[pl.ds(i * L, L)]
        d16 = dst_v[pl.ds(i * L, L)]
        a1 = plsc.load_gather(as_v, [s16])
        a2 = plsc.load_gather(ad_v, [d16])
        e = a1 + a2
        e = jnp.where(e >= 0.0, e, 0.2 * e)
        ee = jnp.exp(e - m_tot)
        eexp_v[pl.ds(i * L, L)] = ee
        plsc.addupdate_scatter(denom_v, [d16], ee)
        return c
    lax.fori_loop(0, EW // L, edge_body, 0)

    pltpu.sync_copy(eexp_v, eexp_hbm.at[w])

    pltpu.sync_copy(denom_v, dsh.at[sid])
    plsc.subcore_barrier()
    base = sid * DSL
    pltpu.sync_copy(dsh.at[0, pl.ds(base, DSL)], red_a)

    def red_body(r, c):
        pltpu.sync_copy(dsh.at[r, pl.ds(base, DSL)], red_b)
        for k in range(DSL // L):
            red_a[pl.ds(k * L, L)] = (red_a[pl.ds(k * L, L)]
                                      + red_b[pl.ds(k * L, L)])
        return c
    lax.fori_loop(1, NS, red_body, 0)
    pltpu.sync_copy(red_a, dpart_hbm.at[cid, pl.ds(base, DSL)])


_sc_edge = pl.kernel(
    _sc_edge_body,
    out_type=(jax.ShapeDtypeStruct((NW, EW), jnp.float32),
              jax.ShapeDtypeStruct((NC, NP), jnp.float32)),
    mesh=_MESH,
    scratch_types=[
        pltpu.VMEM((EW,), jnp.int32),
        pltpu.VMEM((EW,), jnp.int32),
        pltpu.VMEM((N,), jnp.float32),
        pltpu.VMEM((N,), jnp.float32),
        pltpu.VMEM((EW,), jnp.float32),
        pltpu.VMEM((NP,), jnp.float32),
        pltpu.VMEM((DSL,), jnp.float32),
        pltpu.VMEM((DSL,), jnp.float32),
        pltpu.VMEM_SHARED((NS, NP), jnp.float32),
    ],
)


def _sc_msg_body(h_hbm, srcb_hbm, dstb_hbm, eexp_hbm, dpart_hbm, out_hbm,
                 srcb_v, dstb_v, eexp_v, denom_v, tmp_v, alpha_v, rows_v,
                 zrow_v, acc_sh, sem):
    cid = lax.axis_index("c")
    sid = lax.axis_index("s")
    w = cid * NS + sid

    zv = jnp.zeros((L,), jnp.float32)
    for j in range(ZR):
        for k in range(D // L):
            zrow_v[j, pl.ds(k * L, L)] = zv
    base_rows = sid * ORT
    for r in range(ORT // ZR):
        pltpu.sync_copy(zrow_v, acc_sh.at[pl.ds(base_rows + r * ZR, ZR)])

    pltpu.sync_copy(srcb_hbm.at[w], srcb_v)
    pltpu.sync_copy(dstb_hbm.at[w], dstb_v)
    pltpu.sync_copy(eexp_hbm.at[w], eexp_v)

    pltpu.sync_copy(dpart_hbm.at[0], denom_v)
    pltpu.sync_copy(dpart_hbm.at[1], tmp_v)

    def dadd(i, c):
        denom_v[pl.ds(i * L, L)] = (denom_v[pl.ds(i * L, L)]
                                    + tmp_v[pl.ds(i * L, L)])
        return c
    lax.fori_loop(0, NP // L, dadd, 0)

    plsc.subcore_barrier()

    def blk(j, c):
        pltpu.async_copy(h_hbm.at[srcb_v.at[j]], rows_v, sem).wait()
        for k in range(KB // L):
            d16 = dstb_v[j, pl.ds(k * L, L)]
            ee = eexp_v[pl.ds(j * KB + k * L, L)]
            dsum = plsc.load_gather(denom_v, [d16])
            alpha_v[pl.ds(k * L, L)] = ee / (dsum + 1e-16)

        def rowscale(r, c2):
            a = alpha_v[r]
            for k in range(D // L):
                rows_v[r, pl.ds(k * L, L)] = rows_v[r, pl.ds(k * L, L)] * a
            return c2
        lax.fori_loop(0, KB, rowscale, 0)
        pltpu.sync_copy(rows_v, acc_sh.at[dstb_v.at[j]], add=True)
        return c
    lax.fori_loop(0, NBLK, blk, 0)

    plsc.subcore_barrier()
    pltpu.sync_copy(acc_sh.at[pl.ds(base_rows, ORT)],
                    out_hbm.at[cid, pl.ds(base_rows, ORT)])


_sc_msg = pl.kernel(
    _sc_msg_body,
    out_type=jax.ShapeDtypeStruct((NC, N, D), jnp.float32),
    mesh=_MESH,
    scratch_types=[
        pltpu.VMEM((NBLK, KB), jnp.int32),
        pltpu.VMEM((NBLK, KB), jnp.int32),
        pltpu.VMEM((EW,), jnp.float32),
        pltpu.VMEM((NP,), jnp.float32),
        pltpu.VMEM((NP,), jnp.float32),
        pltpu.VMEM((KB,), jnp.float32),
        pltpu.VMEM((KB, D), jnp.float32),
        pltpu.VMEM((ZR, D), jnp.float32),
        pltpu.VMEM_SHARED((N, D), jnp.float32),
        pltpu.SemaphoreType.DMA,
    ],
)


def kernel(x, edge_index, W1, a_src1, a_dst1, b1, W2, a_src2, a_dst2, b2):
    src = edge_index[0].reshape(NW, EW)
    dst = edge_index[1].reshape(NW, EW)
    srcb = edge_index[0].reshape(NW, NBLK, KB)
    dstb = edge_index[1].reshape(NW, NBLK, KB)

    h1, as1, ad1 = _tc_head1(x, W1, a_src1, a_dst1)
    eexp1, dpart1 = _sc_edge(src, dst, as1, ad1)
    p1 = _sc_msg(h1, srcb, dstb, eexp1, dpart1)

    h2, as2, ad2 = _tc_head2(p1, b1, W2, a_src2, a_dst2)
    eexp2, dpart2 = _sc_edge(src, dst, as2, ad2)
    p2 = _sc_msg(h2, srcb, dstb, eexp2, dpart2)

    return _tc_out(p2, b2)


# speedup vs baseline: 18.2837x; 18.2837x over previous
import jax
import jax.numpy as jnp
from jax import lax
from jax.experimental import pallas as pl
from jax.experimental.pallas import tpu as pltpu
from jax.experimental.pallas import tpu_sc as plsc

N = 10000
E = 320000
D = 128
NC = 2
NS = 16
NW = NC * NS
L = 16
EW = E // NW
KB = 80
NBLK = EW // KB
NP = 10240
DSL = NP // NS
ORT = 624
OREM = N - NS * ORT
ZR = 24

_MESH = plsc.VectorSubcoreMesh(core_axis_name="c", subcore_axis_name="s")
_SC_PARAMS = pltpu.CompilerParams(needs_layout_passes=False)


def _tc_head1_body(x_ref, w_ref, asr_ref, adr_ref, h_ref, as_ref, ad_ref):
    h = jnp.dot(x_ref[...], w_ref[...], preferred_element_type=jnp.float32)
    h_ref[...] = h
    as_ref[...] = jnp.sum(h * asr_ref[...][None, :], axis=1, keepdims=True)
    ad_ref[...] = jnp.sum(h * adr_ref[...][None, :], axis=1, keepdims=True)


def _tc_head2_body(p_ref, b_ref, w_ref, asr_ref, adr_ref, h_ref, as_ref,
                   ad_ref):
    hid = jnp.maximum(p_ref[0] + p_ref[1] + b_ref[...][None, :], 0.0)
    h = jnp.dot(hid, w_ref[...], preferred_element_type=jnp.float32)
    h_ref[...] = h
    as_ref[...] = jnp.sum(h * asr_ref[...][None, :], axis=1, keepdims=True)
    ad_ref[...] = jnp.sum(h * adr_ref[...][None, :], axis=1, keepdims=True)


def _tc_out_body(p_ref, b_ref, o_ref):
    o = p_ref[0] + p_ref[1] + b_ref[...][None, :]
    m = jnp.max(o, axis=1, keepdims=True)
    ex = jnp.exp(o - m)
    o_ref[...] = o - m - jnp.log(jnp.sum(ex, axis=1, keepdims=True))


_BN = 2000

_head_out = [
    jax.ShapeDtypeStruct((N, D), jnp.float32),
    jax.ShapeDtypeStruct((N, 1), jnp.float32),
    jax.ShapeDtypeStruct((N, 1), jnp.float32),
]
_head_out_specs = [
    pl.BlockSpec((_BN, D), lambda i: (i, 0)),
    pl.BlockSpec((_BN, 1), lambda i: (i, 0)),
    pl.BlockSpec((_BN, 1), lambda i: (i, 0)),
]
_full_mat = pl.BlockSpec((D, D), lambda i: (0, 0))
_full_vec = pl.BlockSpec((D,), lambda i: (0,))


def _tc_head1(x, W, a_src, a_dst):
    return pl.pallas_call(
        _tc_head1_body,
        grid=(N // _BN,),
        in_specs=[pl.BlockSpec((_BN, D), lambda i: (i, 0)), _full_mat,
                  _full_vec, _full_vec],
        out_specs=_head_out_specs,
        out_shape=_head_out,
    )(x, W, a_src, a_dst)


def _tc_head2(p, b, W, a_src, a_dst):
    return pl.pallas_call(
        _tc_head2_body,
        grid=(N // _BN,),
        in_specs=[pl.BlockSpec((NC, _BN, D), lambda i: (0, i, 0)), _full_vec,
                  _full_mat, _full_vec, _full_vec],
        out_specs=_head_out_specs,
        out_shape=_head_out,
    )(p, b, W, a_src, a_dst)


def _tc_out(p, b):
    return pl.pallas_call(
        _tc_out_body,
        grid=(N // _BN,),
        in_specs=[pl.BlockSpec((NC, _BN, D), lambda i: (0, i, 0)), _full_vec],
        out_specs=pl.BlockSpec((_BN, D), lambda i: (i, 0)),
        out_shape=jax.ShapeDtypeStruct((N, D), jnp.float32),
    )(p, b)


def _sc_edge_body(src_hbm, dst_hbm, as_hbm, ad_hbm, eexp_hbm, dpart_hbm,
                  src_v, dst_v, as_v, ad_v, eexp_v, denom_v, red_a, red_b,
                  dsh):
    cid = lax.axis_index("c")
    sid = lax.axis_index("s")
    w = cid * NS + sid

    pltpu.sync_copy(src_hbm.at[w], src_v)
    pltpu.sync_copy(dst_hbm.at[w], dst_v)
    pltpu.sync_copy(as_hbm, as_v)
    pltpu.sync_copy(ad_hbm, ad_v)

    zv = jnp.zeros((L,), jnp.float32)

    def zero_body(i, c):
        denom_v[pl.ds(i * L, L)] = zv
        return c
    lax.fori_loop(0, NP // L, zero_body, 0)

    neg = jnp.full((L,), -1e30, jnp.float32)

    def max_body(i, carry):
        ms, md = carry
        ms = jnp.maximum(ms, as_v[pl.ds(i * L, L)])
        md = jnp.maximum(md, ad_v[pl.ds(i * L, L)])
        return ms, md
    ms, md = lax.fori_loop(0, N // L, max_body, (neg, neg))

    lanes = lax.iota(jnp.int32, L)

    def allmax(v):
        for step in (8, 4, 2, 1):
            red_a[pl.ds(0, L)] = v
            v = jnp.maximum(v, plsc.load_gather(red_a, [lanes ^ step]))
        return v

    m16 = allmax(ms) + allmax(md)
    m_tot = jnp.where(m16 >= 0.0, m16, 0.2 * m16)

    def edge_body(i, c):
        s16 = src_v[pl.ds(i * L, L)]
        d16 = dst_v[pl.ds(i * L, L)]
        a1 = plsc.load_gather(as_v, [s16])
        a2 = plsc.load_gather(ad_v, [d16])
        e = a1 + a2
        e = jnp.where(e >= 0.0, e, 0.2 * e)
        ee = jnp.exp(e - m_tot)
        eexp_v[pl.ds(i * L, L)] = ee
        plsc.addupdate_scatter(denom_v, [d16], ee)
        return c
    lax.fori_loop(0, EW // L, edge_body, 0)

    pltpu.sync_copy(eexp_v, eexp_hbm.at[w])

    pltpu.sync_copy(denom_v, dsh.at[sid])
    plsc.subcore_barrier()
    base = sid * DSL
    pltpu.sync_copy(dsh.at[0, pl.ds(base, DSL)], red_a)

    def red_body(r, c):
        pltpu.sync_copy(dsh.at[r, pl.ds(base, DSL)], red_b)
        for k in range(DSL // L):
            red_a[pl.ds(k * L, L)] = (red_a[pl.ds(k * L, L)]
                                      + red_b[pl.ds(k * L, L)])
        return c
    lax.fori_loop(1, NS, red_body, 0)
    pltpu.sync_copy(red_a, dpart_hbm.at[cid, pl.ds(base, DSL)])


_sc_edge = pl.kernel(
    _sc_edge_body,
    out_type=(jax.ShapeDtypeStruct((NW, EW), jnp.float32),
              jax.ShapeDtypeStruct((NC, NP), jnp.float32)),
    mesh=_MESH,
    compiler_params=_SC_PARAMS,
    scratch_types=[
        pltpu.VMEM((EW,), jnp.int32),
        pltpu.VMEM((EW,), jnp.int32),
        pltpu.VMEM((N,), jnp.float32),
        pltpu.VMEM((N,), jnp.float32),
        pltpu.VMEM((EW,), jnp.float32),
        pltpu.VMEM((NP,), jnp.float32),
        pltpu.VMEM((DSL,), jnp.float32),
        pltpu.VMEM((DSL,), jnp.float32),
        pltpu.VMEM_SHARED((NS, NP), jnp.float32),
    ],
)


def _sc_msg_body(h_hbm, srcb_hbm, dstb_hbm, eexp_hbm, dpart_hbm, out_hbm,
                 sidx_v, didx_v, eexpb_v, denom_v, tmp_v, alpha_v, rows_v,
                 acc_sh, sem):
    cid = lax.axis_index("c")
    sid = lax.axis_index("s")
    w = cid * NS + sid

    zv = jnp.zeros((L,), jnp.float32)
    for j in range(ZR):
        for k in range(D // L):
            rows_v[j, pl.ds(k * L, L)] = zv
    base_rows = sid * ORT
    for r in range(ORT // ZR):
        pltpu.sync_copy(rows_v.at[pl.ds(0, ZR)],
                        acc_sh.at[pl.ds(base_rows + r * ZR, ZR)])

    @pl.when(sid == NS - 1)
    def _zero_tail():
        pltpu.sync_copy(rows_v.at[pl.ds(0, OREM)],
                        acc_sh.at[pl.ds(NS * ORT, OREM)])

    pltpu.sync_copy(dpart_hbm.at[0], denom_v)
    pltpu.sync_copy(dpart_hbm.at[1], tmp_v)

    def dadd(i, c):
        denom_v[pl.ds(i * L, L)] = (denom_v[pl.ds(i * L, L)]
                                    + tmp_v[pl.ds(i * L, L)])
        return c
    lax.fori_loop(0, NP // L, dadd, 0)

    plsc.subcore_barrier()

    def blk(j, c):
        pltpu.sync_copy(srcb_hbm.at[w, j], sidx_v)
        pltpu.sync_copy(dstb_hbm.at[w, j], didx_v.at[0])
        pltpu.sync_copy(eexp_hbm.at[w, j], eexpb_v)
        pltpu.async_copy(h_hbm.at[sidx_v], rows_v, sem).wait()
        for k in range(KB // L):
            d16 = didx_v[0, pl.ds(k * L, L)]
            ee = eexpb_v[pl.ds(k * L, L)]
            dsum = plsc.load_gather(denom_v, [d16])
            alpha_v[pl.ds(0, L)] = ee / (dsum + 1e-16)
            for rr in range(L):
                r = k * L + rr
                a16 = plsc.load_gather(alpha_v, [jnp.full((L,), rr,
                                                          jnp.int32)])
                for q in range(D // L):
                    rows_v[r, pl.ds(q * L, L)] = (
                        rows_v[r, pl.ds(q * L, L)] * a16)
        pltpu.sync_copy(rows_v, acc_sh.at[didx_v.at[0]], add=True)
        return c
    lax.fori_loop(0, NBLK, blk, 0)

    plsc.subcore_barrier()
    pltpu.sync_copy(acc_sh.at[pl.ds(base_rows, ORT)],
                    out_hbm.at[cid, pl.ds(base_rows, ORT)])

    @pl.when(sid == NS - 1)
    def _copy_tail():
        pltpu.sync_copy(acc_sh.at[pl.ds(NS * ORT, OREM)],
                        out_hbm.at[cid, pl.ds(NS * ORT, OREM)])


_sc_msg = pl.kernel(
    _sc_msg_body,
    out_type=jax.ShapeDtypeStruct((NC, N, D), jnp.float32),
    mesh=_MESH,
    compiler_params=_SC_PARAMS,
    scratch_types=[
        pltpu.VMEM((KB,), jnp.int32),
        pltpu.VMEM((1, KB), jnp.int32),
        pltpu.VMEM((KB,), jnp.float32),
        pltpu.VMEM((NP,), jnp.float32),
        pltpu.VMEM((NP,), jnp.float32),
        pltpu.VMEM((L,), jnp.float32),
        pltpu.VMEM((KB, D), jnp.float32),
        pltpu.VMEM_SHARED((N, D), jnp.float32),
        pltpu.SemaphoreType.DMA,
    ],
)


def kernel(x, edge_index, W1, a_src1, a_dst1, b1, W2, a_src2, a_dst2, b2):
    src = edge_index[0].reshape(NW, EW)
    dst = edge_index[1].reshape(NW, EW)
    srcb = edge_index[0].reshape(NW, NBLK, KB)
    dstb = edge_index[1].reshape(NW, NBLK, KB)

    h1, as1, ad1 = _tc_head1(x, W1, a_src1, a_dst1)
    eexp1, dpart1 = _sc_edge(src, dst, as1.reshape(N), ad1.reshape(N))
    p1 = _sc_msg(h1, srcb, dstb, eexp1.reshape(NW, NBLK, KB), dpart1)

    h2, as2, ad2 = _tc_head2(p1, b1, W2, a_src2, a_dst2)
    eexp2, dpart2 = _sc_edge(src, dst, as2.reshape(N), ad2.reshape(N))
    p2 = _sc_msg(h2, srcb, dstb, eexp2.reshape(NW, NBLK, KB), dpart2)

    return _tc_out(p2, b2)


# speedup vs baseline: 26.4008x; 1.4440x over previous
import jax
import jax.numpy as jnp
from jax import lax
from jax.experimental import pallas as pl
from jax.experimental.pallas import tpu as pltpu
from jax.experimental.pallas import tpu_sc as plsc

N = 10000
E = 320000
D = 128
NC = 2
NS = 16
NW = NC * NS
L = 16
EW = E // NW
KB = 80
NBLK = EW // KB
NP = 10240
DSL = NP // NS
ORT = 624
OREM = N - NS * ORT
ZR = 24

_MESH = plsc.VectorSubcoreMesh(core_axis_name="c", subcore_axis_name="s")
_SC_PARAMS = pltpu.CompilerParams(needs_layout_passes=False)


def _tc_head1_body(x_ref, w_ref, asr_ref, adr_ref, h_ref, as_ref, ad_ref):
    h = jnp.dot(x_ref[...], w_ref[...], preferred_element_type=jnp.float32)
    h_ref[...] = h
    as_ref[...] = jnp.sum(h * asr_ref[...][None, :], axis=1, keepdims=True)
    ad_ref[...] = jnp.sum(h * adr_ref[...][None, :], axis=1, keepdims=True)


def _tc_head2_body(p_ref, b_ref, w_ref, asr_ref, adr_ref, h_ref, as_ref,
                   ad_ref):
    hid = jnp.maximum(p_ref[0] + p_ref[1] + b_ref[...][None, :], 0.0)
    h = jnp.dot(hid, w_ref[...], preferred_element_type=jnp.float32)
    h_ref[...] = h
    as_ref[...] = jnp.sum(h * asr_ref[...][None, :], axis=1, keepdims=True)
    ad_ref[...] = jnp.sum(h * adr_ref[...][None, :], axis=1, keepdims=True)


def _tc_out_body(p_ref, b_ref, o_ref):
    o = p_ref[0] + p_ref[1] + b_ref[...][None, :]
    m = jnp.max(o, axis=1, keepdims=True)
    ex = jnp.exp(o - m)
    o_ref[...] = o - m - jnp.log(jnp.sum(ex, axis=1, keepdims=True))


_BN = 2000

_head_out = [
    jax.ShapeDtypeStruct((N, D), jnp.float32),
    jax.ShapeDtypeStruct((N, 1), jnp.float32),
    jax.ShapeDtypeStruct((N, 1), jnp.float32),
]
_head_out_specs = [
    pl.BlockSpec((_BN, D), lambda i: (i, 0)),
    pl.BlockSpec((_BN, 1), lambda i: (i, 0)),
    pl.BlockSpec((_BN, 1), lambda i: (i, 0)),
]
_full_mat = pl.BlockSpec((D, D), lambda i: (0, 0))
_full_vec = pl.BlockSpec((D,), lambda i: (0,))


def _tc_head1(x, W, a_src, a_dst):
    return pl.pallas_call(
        _tc_head1_body,
        grid=(N // _BN,),
        in_specs=[pl.BlockSpec((_BN, D), lambda i: (i, 0)), _full_mat,
                  _full_vec, _full_vec],
        out_specs=_head_out_specs,
        out_shape=_head_out,
    )(x, W, a_src, a_dst)


def _tc_head2(p, b, W, a_src, a_dst):
    return pl.pallas_call(
        _tc_head2_body,
        grid=(N // _BN,),
        in_specs=[pl.BlockSpec((NC, _BN, D), lambda i: (0, i, 0)), _full_vec,
                  _full_mat, _full_vec, _full_vec],
        out_specs=_head_out_specs,
        out_shape=_head_out,
    )(p, b, W, a_src, a_dst)


def _tc_out(p, b):
    return pl.pallas_call(
        _tc_out_body,
        grid=(N // _BN,),
        in_specs=[pl.BlockSpec((NC, _BN, D), lambda i: (0, i, 0)), _full_vec],
        out_specs=pl.BlockSpec((_BN, D), lambda i: (i, 0)),
        out_shape=jax.ShapeDtypeStruct((N, D), jnp.float32),
    )(p, b)


def _sc_edge_body(src_hbm, dst_hbm, as_hbm, ad_hbm, eexp_hbm, dpart_hbm,
                  src_v, dst_v, as_v, ad_v, eexp_v, denom_v, red_a, red_b,
                  dsh):
    cid = lax.axis_index("c")
    sid = lax.axis_index("s")
    w = cid * NS + sid

    pltpu.sync_copy(src_hbm.at[w], src_v)
    pltpu.sync_copy(dst_hbm.at[w], dst_v)
    pltpu.sync_copy(as_hbm, as_v)
    pltpu.sync_copy(ad_hbm, ad_v)

    zv = jnp.zeros((L,), jnp.float32)

    def zero_body(i, c):
        denom_v[pl.ds(i * L, L)] = zv
        return c
    lax.fori_loop(0, NP // L, zero_body, 0)

    neg = jnp.full((L,), -1e30, jnp.float32)

    def max_body(i, carry):
        ms, md = carry
        ms = jnp.maximum(ms, as_v[pl.ds(i * L, L)])
        md = jnp.maximum(md, ad_v[pl.ds(i * L, L)])
        return ms, md
    ms, md = lax.fori_loop(0, N // L, max_body, (neg, neg))

    lanes = lax.iota(jnp.int32, L)

    def allmax(v):
        for step in (8, 4, 2, 1):
            red_a[pl.ds(0, L)] = v
            v = jnp.maximum(v, plsc.load_gather(red_a, [lanes ^ step]))
        return v

    m16 = allmax(ms) + allmax(md)
    m_tot = jnp.where(m16 >= 0.0, m16, 0.2 * m16)

    def edge_body(i, c):
        s16 = src_v[pl.ds(i * L, L)]
        d16 = dst_v[pl.ds(i * L, L)]
        a1 = plsc.load_gather(as_v, [s16])
        a2 = plsc.load_gather(ad_v, [d16])
        e = a1 + a2
        e = jnp.where(e >= 0.0, e, 0.2 * e)
        ee = jnp.exp(e - m_tot)
        eexp_v[pl.ds(i * L, L)] = ee
        plsc.addupdate_scatter(denom_v, [d16], ee)
        return c
    lax.fori_loop(0, EW // L, edge_body, 0)

    pltpu.sync_copy(eexp_v, eexp_hbm.at[w])

    pltpu.sync_copy(denom_v, dsh.at[sid])
    plsc.subcore_barrier()
    base = sid * DSL
    pltpu.sync_copy(dsh.at[0, pl.ds(base, DSL)], red_a)

    def red_body(r, c):
        pltpu.sync_copy(dsh.at[r, pl.ds(base, DSL)], red_b)
        for k in range(DSL // L):
            red_a[pl.ds(k * L, L)] = (red_a[pl.ds(k * L, L)]
                                      + red_b[pl.ds(k * L, L)])
        return c
    lax.fori_loop(1, NS, red_body, 0)
    pltpu.sync_copy(red_a, dpart_hbm.at[cid, pl.ds(base, DSL)])


_sc_edge = pl.kernel(
    _sc_edge_body,
    out_type=(jax.ShapeDtypeStruct((NW, EW), jnp.float32),
              jax.ShapeDtypeStruct((NC, NP), jnp.float32)),
    mesh=_MESH,
    compiler_params=_SC_PARAMS,
    scratch_types=[
        pltpu.VMEM((EW,), jnp.int32),
        pltpu.VMEM((EW,), jnp.int32),
        pltpu.VMEM((N,), jnp.float32),
        pltpu.VMEM((N,), jnp.float32),
        pltpu.VMEM((EW,), jnp.float32),
        pltpu.VMEM((NP,), jnp.float32),
        pltpu.VMEM((DSL,), jnp.float32),
        pltpu.VMEM((DSL,), jnp.float32),
        pltpu.VMEM_SHARED((NS, NP), jnp.float32),
    ],
)


def _sc_comb_body(eexp_hbm, dst_hbm, dpart_hbm, alpha_hbm,
                  eexp_v, dst_v, rec_v, tmp_v, alpha_v):
    cid = lax.axis_index("c")
    sid = lax.axis_index("s")
    w = cid * NS + sid

    pltpu.sync_copy(eexp_hbm.at[w], eexp_v)
    pltpu.sync_copy(dst_hbm.at[w], dst_v)
    pltpu.sync_copy(dpart_hbm.at[0], rec_v)
    pltpu.sync_copy(dpart_hbm.at[1], tmp_v)

    one = jnp.ones((L,), jnp.float32)

    def rbody(i, c):
        d = rec_v[pl.ds(i * L, L)] + tmp_v[pl.ds(i * L, L)] + 1e-16
        rec_v[pl.ds(i * L, L)] = one / d
        return c
    lax.fori_loop(0, NP // L, rbody, 0)

    def abody(i, c):
        d16 = dst_v[pl.ds(i * L, L)]
        alpha_v[pl.ds(i * L, L)] = (eexp_v[pl.ds(i * L, L)]
                                    * plsc.load_gather(rec_v, [d16]))
        return c
    lax.fori_loop(0, EW // L, abody, 0)

    pltpu.sync_copy(alpha_v, alpha_hbm.at[w])


_sc_comb = pl.kernel(
    _sc_comb_body,
    out_type=jax.ShapeDtypeStruct((NW, EW), jnp.float32),
    mesh=_MESH,
    compiler_params=_SC_PARAMS,
    scratch_types=[
        pltpu.VMEM((EW,), jnp.float32),
        pltpu.VMEM((EW,), jnp.int32),
        pltpu.VMEM((NP,), jnp.float32),
        pltpu.VMEM((NP,), jnp.float32),
        pltpu.VMEM((EW,), jnp.float32),
    ],
)

_SBYTES = KB * D * 4


def _sc_msg_body(h_hbm, pkd_hbm, alpha_hbm, out_hbm,
                 pkd_v, alphab_v, rows_v, acc_sh, gsem, ssem):
    cid = lax.axis_index("c")
    sid = lax.axis_index("s")
    w = cid * NS + sid

    zv = jnp.zeros((L,), jnp.float32)
    for j in range(ZR):
        for k in range(D // L):
            rows_v[0, j, pl.ds(k * L, L)] = zv
    base_rows = sid * ORT
    for r in range(ORT // ZR):
        pltpu.sync_copy(rows_v.at[0, pl.ds(0, ZR)],
                        acc_sh.at[pl.ds(base_rows + r * ZR, ZR)])

    @pl.when(sid == NS - 1)
    def _zero_tail():
        pltpu.sync_copy(rows_v.at[0, pl.ds(0, OREM)],
                        acc_sh.at[pl.ds(NS * ORT, OREM)])

    plsc.subcore_barrier()

    def load_idx(j, s):
        pltpu.sync_copy(pkd_hbm.at[w, j], pkd_v.at[s])
        pltpu.sync_copy(alpha_hbm.at[w, j], alphab_v.at[s])

    def start_gather(s):
        pltpu.async_copy(h_hbm.at[pkd_v.at[s, 0]], rows_v.at[s],
                         gsem.at[s])

    def start_scatter(s):
        pltpu.async_copy(rows_v.at[s], acc_sh.at[pkd_v.at[s, 1]],
                         ssem.at[s], add=True)

    def wait_gather(s):
        pltpu.make_async_copy(h_hbm.at[pkd_v.at[s, 0]], rows_v.at[s],
                              gsem.at[s]).wait()

    def wait_scatter(s):
        pltpu.make_async_copy(rows_v.at[s], acc_sh.at[pkd_v.at[s, 1]],
                              ssem.at[s]).wait()

    def compute(s):
        for k in range(KB // L):
            for rr in range(L):
                r = k * L + rr
                a16 = plsc.load_gather(
                    alphab_v, [jnp.full((L,), s, jnp.int32),
                               jnp.full((L,), r, jnp.int32)])
                for q in range(D // L):
                    rows_v[s, r, pl.ds(q * L, L)] = (
                        rows_v[s, r, pl.ds(q * L, L)] * a16)

    load_idx(0, 0)
    start_gather(0)

    load_idx(1, 1)
    start_gather(1)
    wait_gather(0)
    compute(0)
    start_scatter(0)

    def step(j, c):
        s = j % 2
        t = 1 - s
        wait_scatter(t)
        load_idx(j + 1, t)
        start_gather(t)
        wait_gather(s)
        compute(s)
        start_scatter(s)
        return c
    lax.fori_loop(1, NBLK - 1, step, 0)

    wait_scatter(1)
    wait_gather(0)
    compute(0)
    pltpu.sync_copy(rows_v.at[0], acc_sh.at[pkd_v.at[0, 1]], add=True)

    plsc.subcore_barrier()
    pltpu.sync_copy(acc_sh.at[pl.ds(base_rows, ORT)],
                    out_hbm.at[cid, pl.ds(base_rows, ORT)])

    @pl.when(sid == NS - 1)
    def _copy_tail():
        pltpu.sync_copy(acc_sh.at[pl.ds(NS * ORT, OREM)],
                        out_hbm.at[cid, pl.ds(NS * ORT, OREM)])


_sc_msg = pl.kernel(
    _sc_msg_body,
    out_type=jax.ShapeDtypeStruct((NC, N, D), jnp.float32),
    mesh=_MESH,
    compiler_params=_SC_PARAMS,
    scratch_types=[
        pltpu.VMEM((2, 2, KB), jnp.int32),
        pltpu.VMEM((2, KB), jnp.float32),
        pltpu.VMEM((2, KB, D), jnp.float32),
        pltpu.VMEM_SHARED((N, D), jnp.float32),
        pltpu.SemaphoreType.DMA((2,)),
        pltpu.SemaphoreType.DMA((2,)),
    ],
)


def kernel(x, edge_index, W1, a_src1, a_dst1, b1, W2, a_src2, a_dst2, b2):
    src = edge_index[0].reshape(NW, EW)
    dst = edge_index[1].reshape(NW, EW)
    srcb = edge_index[0].reshape(NW, NBLK, KB)
    dstb = edge_index[1].reshape(NW, NBLK, KB)
    pkd = jnp.stack([srcb, dstb], axis=2)

    h1, as1, ad1 = _tc_head1(x, W1, a_src1, a_dst1)
    eexp1, dpart1 = _sc_edge(src, dst, as1.reshape(N), ad1.reshape(N))
    alpha1 = _sc_comb(eexp1, dst, dpart1)
    p1 = _sc_msg(h1, pkd, alpha1.reshape(NW, NBLK, KB))

    h2, as2, ad2 = _tc_head2(p1, b1, W2, a_src2, a_dst2)
    eexp2, dpart2 = _sc_edge(src, dst, as2.reshape(N), ad2.reshape(N))
    alpha2 = _sc_comb(eexp2, dst, dpart2)
    p2 = _sc_msg(h2, pkd, alpha2.reshape(NW, NBLK, KB))

    return _tc_out(p2, b2)


# speedup vs baseline: 35.4242x; 1.3418x over previous
import jax
import jax.numpy as jnp
from jax import lax
from jax.experimental import pallas as pl
from jax.experimental.pallas import tpu as pltpu
from jax.experimental.pallas import tpu_sc as plsc

N = 10000
E = 320000
D = 128
NC = 2
NS = 16
NW = NC * NS
L = 16
EW = E // NW
KB = 80
NBLK = EW // KB
NP = 10240
DSL = NP // NS
ORT = 624
OREM = N - NS * ORT
ZR = 24

_MESH = plsc.VectorSubcoreMesh(core_axis_name="c", subcore_axis_name="s")
_SC_PARAMS = pltpu.CompilerParams(needs_layout_passes=False)


def _tc_head1_body(x_ref, w_ref, asr_ref, adr_ref, h_ref, as_ref, ad_ref):
    h = jnp.dot(x_ref[...], w_ref[...], preferred_element_type=jnp.float32)
    h_ref[...] = h
    as_ref[...] = jnp.sum(h * asr_ref[...][None, :], axis=1, keepdims=True)
    ad_ref[...] = jnp.sum(h * adr_ref[...][None, :], axis=1, keepdims=True)


def _tc_head2_body(p_ref, b_ref, w_ref, asr_ref, adr_ref, h_ref, as_ref,
                   ad_ref):
    hid = jnp.maximum(p_ref[0] + p_ref[1] + b_ref[...][None, :], 0.0)
    h = jnp.dot(hid, w_ref[...], preferred_element_type=jnp.float32)
    h_ref[...] = h
    as_ref[...] = jnp.sum(h * asr_ref[...][None, :], axis=1, keepdims=True)
    ad_ref[...] = jnp.sum(h * adr_ref[...][None, :], axis=1, keepdims=True)


def _tc_out_body(p_ref, b_ref, o_ref):
    o = p_ref[0] + p_ref[1] + b_ref[...][None, :]
    m = jnp.max(o, axis=1, keepdims=True)
    ex = jnp.exp(o - m)
    o_ref[...] = o - m - jnp.log(jnp.sum(ex, axis=1, keepdims=True))


_BN = 2000

_head_out = [
    jax.ShapeDtypeStruct((N, D), jnp.float32),
    jax.ShapeDtypeStruct((N, 1), jnp.float32),
    jax.ShapeDtypeStruct((N, 1), jnp.float32),
]
_head_out_specs = [
    pl.BlockSpec((_BN, D), lambda i: (i, 0)),
    pl.BlockSpec((_BN, 1), lambda i: (i, 0)),
    pl.BlockSpec((_BN, 1), lambda i: (i, 0)),
]
_full_mat = pl.BlockSpec((D, D), lambda i: (0, 0))
_full_vec = pl.BlockSpec((D,), lambda i: (0,))


def _tc_head1(x, W, a_src, a_dst):
    return pl.pallas_call(
        _tc_head1_body,
        grid=(N // _BN,),
        in_specs=[pl.BlockSpec((_BN, D), lambda i: (i, 0)), _full_mat,
                  _full_vec, _full_vec],
        out_specs=_head_out_specs,
        out_shape=_head_out,
    )(x, W, a_src, a_dst)


def _tc_head2(p, b, W, a_src, a_dst):
    return pl.pallas_call(
        _tc_head2_body,
        grid=(N // _BN,),
        in_specs=[pl.BlockSpec((NC, _BN, D), lambda i: (0, i, 0)), _full_vec,
                  _full_mat, _full_vec, _full_vec],
        out_specs=_head_out_specs,
        out_shape=_head_out,
    )(p, b, W, a_src, a_dst)


def _tc_out(p, b):
    return pl.pallas_call(
        _tc_out_body,
        grid=(N // _BN,),
        in_specs=[pl.BlockSpec((NC, _BN, D), lambda i: (0, i, 0)), _full_vec],
        out_specs=pl.BlockSpec((_BN, D), lambda i: (i, 0)),
        out_shape=jax.ShapeDtypeStruct((N, D), jnp.float32),
    )(p, b)


def _sc_edge_body(src_hbm, dst_hbm, as_hbm, ad_hbm, eexp_hbm, dpart_hbm,
                  src_v, dst_v, as_v, ad_v, eexp_v, denom_v, red_a, red_b,
                  dsh):
    cid = lax.axis_index("c")
    sid = lax.axis_index("s")
    w = cid * NS + sid

    pltpu.sync_copy(src_hbm.at[w], src_v)
    pltpu.sync_copy(dst_hbm.at[w], dst_v)
    pltpu.sync_copy(as_hbm, as_v)
    pltpu.sync_copy(ad_hbm, ad_v)

    zv = jnp.zeros((L,), jnp.float32)

    def zero_body(i, c):
        denom_v[pl.ds(i * L, L)] = zv
        return c
    lax.fori_loop(0, NP // L, zero_body, 0)

    neg = jnp.full((L,), -1e30, jnp.float32)

    def max_body(i, carry):
        ms, md = carry
        ms = jnp.maximum(ms, as_v[pl.ds(i * L, L)])
        md = jnp.maximum(md, ad_v[pl.ds(i * L, L)])
        return ms, md
    ms, md = lax.fori_loop(0, N // L, max_body, (neg, neg))

    lanes = lax.iota(jnp.int32, L)

    def allmax(v):
        for step in (8, 4, 2, 1):
            red_a[pl.ds(0, L)] = v
            v = jnp.maximum(v, plsc.load_gather(red_a, [lanes ^ step]))
        return v

    m16 = allmax(ms) + allmax(md)
    m_tot = jnp.where(m16 >= 0.0, m16, 0.2 * m16)

    def edge_body(i, c):
        s16 = src_v[pl.ds(i * L, L)]
        d16 = dst_v[pl.ds(i * L, L)]
        a1 = plsc.load_gather(as_v, [s16])
        a2 = plsc.load_gather(ad_v, [d16])
        e = a1 + a2
        e = jnp.where(e >= 0.0, e, 0.2 * e)
        ee = jnp.exp(e - m_tot)
        eexp_v[pl.ds(i * L, L)] = ee
        plsc.addupdate_scatter(denom_v, [d16], ee)
        return c
    lax.fori_loop(0, EW // L, edge_body, 0)

    pltpu.sync_copy(eexp_v, eexp_hbm.at[w])

    pltpu.sync_copy(denom_v, dsh.at[sid])
    plsc.subcore_barrier()
    base = sid * DSL
    pltpu.sync_copy(dsh.at[0, pl.ds(base, DSL)], red_a)

    def red_body(r, c):
        pltpu.sync_copy(dsh.at[r, pl.ds(base, DSL)], red_b)
        for k in range(DSL // L):
            red_a[pl.ds(k * L, L)] = (red_a[pl.ds(k * L, L)]
                                      + red_b[pl.ds(k * L, L)])
        return c
    lax.fori_loop(1, NS, red_body, 0)
    pltpu.sync_copy(red_a, dpart_hbm.at[cid, pl.ds(base, DSL)])


_sc_edge = pl.kernel(
    _sc_edge_body,
    out_type=(jax.ShapeDtypeStruct((NW, EW), jnp.float32),
              jax.ShapeDtypeStruct((NC, NP), jnp.float32)),
    mesh=_MESH,
    compiler_params=_SC_PARAMS,
    scratch_types=[
        pltpu.VMEM((EW,), jnp.int32),
        pltpu.VMEM((EW,), jnp.int32),
        pltpu.VMEM((N,), jnp.float32),
        pltpu.VMEM((N,), jnp.float32),
        pltpu.VMEM((EW,), jnp.float32),
        pltpu.VMEM((NP,), jnp.float32),
        pltpu.VMEM((DSL,), jnp.float32),
        pltpu.VMEM((DSL,), jnp.float32),
        pltpu.VMEM_SHARED((NS, NP), jnp.float32),
    ],
)


def _sc_comb_body(eexp_hbm, dst_hbm, dpart_hbm, pkd_hbm, pkd3_hbm,
                  eexp_v, dst_v, rec_v, tmp_v, pkd2_v, pkd3_v):
    cid = lax.axis_index("c")
    sid = lax.axis_index("s")
    w = cid * NS + sid

    pltpu.sync_copy(eexp_hbm.at[w], eexp_v)
    pltpu.sync_copy(dst_hbm.at[w], dst_v)
    pltpu.sync_copy(dpart_hbm.at[0], rec_v)
    pltpu.sync_copy(dpart_hbm.at[1], tmp_v)
    pltpu.sync_copy(pkd_hbm.at[w], pkd2_v)

    one = jnp.ones((L,), jnp.float32)

    def rbody(i, c):
        d = rec_v[pl.ds(i * L, L)] + tmp_v[pl.ds(i * L, L)] + 1e-16
        rec_v[pl.ds(i * L, L)] = one / d
        return c
    lax.fori_loop(0, NP // L, rbody, 0)

    def abody(j, c):
        for kk in range(KB // L):
            off = j * KB + kk * L
            d16 = dst_v[pl.ds(off, L)]
            a = eexp_v[pl.ds(off, L)] * plsc.load_gather(rec_v, [d16])
            pkd3_v[j, pl.ds(kk * L, L)] = pkd2_v[j, pl.ds(kk * L, L)]
            pkd3_v[j, pl.ds(KB + kk * L, L)] = (
                pkd2_v[j, pl.ds(KB + kk * L, L)])
            pkd3_v[j, pl.ds(2 * KB + kk * L, L)] = plsc.bitcast(a, jnp.int32)
        return c
    lax.fori_loop(0, NBLK, abody, 0)

    pltpu.sync_copy(pkd3_v, pkd3_hbm.at[w])


_sc_comb = pl.kernel(
    _sc_comb_body,
    out_type=jax.ShapeDtypeStruct((NW, NBLK, 3 * KB), jnp.int32),
    mesh=_MESH,
    compiler_params=_SC_PARAMS,
    scratch_types=[
        pltpu.VMEM((EW,), jnp.float32),
        pltpu.VMEM((EW,), jnp.int32),
        pltpu.VMEM((NP,), jnp.float32),
        pltpu.VMEM((NP,), jnp.float32),
        pltpu.VMEM((NBLK, 2 * KB), jnp.int32),
        pltpu.VMEM((NBLK, 3 * KB), jnp.int32),
    ],
)

_SBYTES = KB * D * 4


def _sc_msg_body(h_hbm, pkd3_hbm, out_hbm,
                 pkd3_v, sdix_v, rows_v, acc_sh, gsem, ssem):
    cid = lax.axis_index("c")
    sid = lax.axis_index("s")
    w = cid * NS + sid

    zv = jnp.zeros((L,), jnp.float32)
    for j in range(ZR):
        for k in range(D // L):
            rows_v[0, j, pl.ds(k * L, L)] = zv
    base_rows = sid * ORT
    for r in range(ORT // ZR):
        pltpu.sync_copy(rows_v.at[0, pl.ds(0, ZR)],
                        acc_sh.at[pl.ds(base_rows + r * ZR, ZR)])

    @pl.when(sid == NS - 1)
    def _zero_tail():
        pltpu.sync_copy(rows_v.at[0, pl.ds(0, OREM)],
                        acc_sh.at[pl.ds(NS * ORT, OREM)])

    plsc.subcore_barrier()

    def load_idx(j, s):
        pltpu.sync_copy(pkd3_hbm.at[w, j], pkd3_v.at[s])

    def start_gather(s):
        pltpu.async_copy(h_hbm.at[pkd3_v.at[s, pl.ds(0, KB)]],
                         rows_v.at[s], gsem.at[s])

    def start_scatter(s):
        pltpu.async_copy(rows_v.at[s], acc_sh.at[sdix_v.at[s]],
                         ssem.at[s], add=True)

    def wait_gather(s):
        pltpu.make_async_copy(h_hbm.at[pkd3_v.at[s, pl.ds(0, KB)]],
                              rows_v.at[s], gsem.at[s]).wait()

    def wait_scatter(s):
        pltpu.make_async_copy(rows_v.at[s], acc_sh.at[sdix_v.at[s]],
                              ssem.at[s]).wait()

    def compute(s):
        for k in range(KB // L):
            sdix_v[s, pl.ds(k * L, L)] = pkd3_v[s, pl.ds(KB + k * L, L)]
        for k in range(KB // L):
            for rr in range(L):
                r = k * L + rr
                a16 = plsc.bitcast(
                    plsc.load_gather(
                        pkd3_v, [jnp.full((L,), s, jnp.int32),
                                 jnp.full((L,), 2 * KB + r, jnp.int32)]),
                    jnp.float32)
                for q in range(D // L):
                    rows_v[s, r, pl.ds(q * L, L)] = (
                        rows_v[s, r, pl.ds(q * L, L)] * a16)

    load_idx(0, 0)
    start_gather(0)

    load_idx(1, 1)
    start_gather(1)
    wait_gather(0)
    compute(0)
    start_scatter(0)

    def step(j, c):
        s = j % 2
        t = 1 - s
        load_idx(j + 1, t)
        wait_scatter(t)
        start_gather(t)
        wait_gather(s)
        compute(s)
        start_scatter(s)
        return c
    lax.fori_loop(1, NBLK - 1, step, 0)

    wait_scatter(1)
    wait_gather(0)
    compute(0)
    pltpu.sync_copy(rows_v.at[0], acc_sh.at[sdix_v.at[0]], add=True)

    plsc.subcore_barrier()
    pltpu.sync_copy(acc_sh.at[pl.ds(base_rows, ORT)],
                    out_hbm.at[cid, pl.ds(base_rows, ORT)])

    @pl.when(sid == NS - 1)
    def _copy_tail():
        pltpu.sync_copy(acc_sh.at[pl.ds(NS * ORT, OREM)],
                        out_hbm.at[cid, pl.ds(NS * ORT, OREM)])


_sc_msg = pl.kernel(
    _sc_msg_body,
    out_type=jax.ShapeDtypeStruct((NC, N, D), jnp.float32),
    mesh=_MESH,
    compiler_params=_SC_PARAMS,
    scratch_types=[
        pltpu.VMEM((2, 3 * KB), jnp.int32),
        pltpu.VMEM((2, KB), jnp.int32),
        pltpu.VMEM((2, KB, D), jnp.float32),
        pltpu.VMEM_SHARED((N, D), jnp.float32),
        pltpu.SemaphoreType.DMA((2,)),
        pltpu.SemaphoreType.DMA((2,)),
    ],
)


def kernel(x, edge_index, W1, a_src1, a_dst1, b1, W2, a_src2, a_dst2, b2):
    src = edge_index[0].reshape(NW, EW)
    dst = edge_index[1].reshape(NW, EW)
    srcb = edge_index[0].reshape(NW, NBLK, KB)
    dstb = edge_index[1].reshape(NW, NBLK, KB)
    pkd = jnp.concatenate([srcb, dstb], axis=2)

    h1, as1, ad1 = _tc_head1(x, W1, a_src1, a_dst1)
    eexp1, dpart1 = _sc_edge(src, dst, as1.reshape(N), ad1.reshape(N))
    pkd3_1 = _sc_comb(eexp1, dst, dpart1, pkd)
    p1 = _sc_msg(h1, pkd3_1)

    h2, as2, ad2 = _tc_head2(p1, b1, W2, a_src2, a_dst2)
    eexp2, dpart2 = _sc_edge(src, dst, as2.reshape(N), ad2.reshape(N))
    pkd3_2 = _sc_comb(eexp2, dst, dpart2, pkd)
    p2 = _sc_msg(h2, pkd3_2)

    return _tc_out(p2, b2)


# speedup vs baseline: 38.0460x; 1.0740x over previous
import jax
import jax.numpy as jnp
from jax import lax
from jax.experimental import pallas as pl
from jax.experimental.pallas import tpu as pltpu
from jax.experimental.pallas import tpu_sc as plsc

N = 10000
E = 320000
D = 128
NC = 2
NS = 16
NW = NC * NS
L = 16
EW = E // NW
KB = 80
NBLK = EW // KB
NP = 10240
DSL = NP // NS
ORT = 624
OREM = N - NS * ORT
ZR = 24

_MESH = plsc.VectorSubcoreMesh(core_axis_name="c", subcore_axis_name="s")
_SC_PARAMS = pltpu.CompilerParams(needs_layout_passes=False)


def _tc_head1_body(x_ref, w_ref, asr_ref, adr_ref, h_ref, as_ref, ad_ref):
    h = jnp.dot(x_ref[...], w_ref[...], preferred_element_type=jnp.float32)
    h_ref[...] = h
    as_ref[...] = jnp.sum(h * asr_ref[...][None, :], axis=1, keepdims=True)
    ad_ref[...] = jnp.sum(h * adr_ref[...][None, :], axis=1, keepdims=True)


def _tc_head2_body(p_ref, b_ref, w_ref, asr_ref, adr_ref, h_ref, as_ref,
                   ad_ref):
    hid = jnp.maximum(p_ref[0] + p_ref[1] + b_ref[...][None, :], 0.0)
    h = jnp.dot(hid, w_ref[...], preferred_element_type=jnp.float32)
    h_ref[...] = h
    as_ref[...] = jnp.sum(h * asr_ref[...][None, :], axis=1, keepdims=True)
    ad_ref[...] = jnp.sum(h * adr_ref[...][None, :], axis=1, keepdims=True)


def _tc_out_body(p_ref, b_ref, o_ref):
    o = p_ref[0] + p_ref[1] + b_ref[...][None, :]
    m = jnp.max(o, axis=1, keepdims=True)
    ex = jnp.exp(o - m)
    o_ref[...] = o - m - jnp.log(jnp.sum(ex, axis=1, keepdims=True))


_BN = 2000

_head_out = [
    jax.ShapeDtypeStruct((N, D), jnp.float32),
    jax.ShapeDtypeStruct((N, 1), jnp.float32),
    jax.ShapeDtypeStruct((N, 1), jnp.float32),
]
_head_out_specs = [
    pl.BlockSpec((_BN, D), lambda i: (i, 0)),
    pl.BlockSpec((_BN, 1), lambda i: (i, 0)),
    pl.BlockSpec((_BN, 1), lambda i: (i, 0)),
]
_full_mat = pl.BlockSpec((D, D), lambda i: (0, 0))
_full_vec = pl.BlockSpec((D,), lambda i: (0,))


def _tc_head1(x, W, a_src, a_dst):
    return pl.pallas_call(
        _tc_head1_body,
        grid=(N // _BN,),
        in_specs=[pl.BlockSpec((_BN, D), lambda i: (i, 0)), _full_mat,
                  _full_vec, _full_vec],
        out_specs=_head_out_specs,
        out_shape=_head_out,
    )(x, W, a_src, a_dst)


def _tc_head2(p, b, W, a_src, a_dst):
    return pl.pallas_call(
        _tc_head2_body,
        grid=(N // _BN,),
        in_specs=[pl.BlockSpec((NC, _BN, D), lambda i: (0, i, 0)), _full_vec,
                  _full_mat, _full_vec, _full_vec],
        out_specs=_head_out_specs,
        out_shape=_head_out,
    )(p, b, W, a_src, a_dst)


def _tc_out(p, b):
    return pl.pallas_call(
        _tc_out_body,
        grid=(N // _BN,),
        in_specs=[pl.BlockSpec((NC, _BN, D), lambda i: (0, i, 0)), _full_vec],
        out_specs=pl.BlockSpec((_BN, D), lambda i: (i, 0)),
        out_shape=jax.ShapeDtypeStruct((N, D), jnp.float32),
    )(p, b)


def _sc_edge_body(pkd2_hbm, as_hbm, ad_hbm, pkd3_hbm, dpart_hbm,
                  pkd2_v, pkd3_v, as_v, ad_v, denom_v, red_a, red_b,
                  dsh):
    cid = lax.axis_index("c")
    sid = lax.axis_index("s")
    w = cid * NS + sid

    pltpu.sync_copy(pkd2_hbm.at[w], pkd2_v)
    pltpu.sync_copy(as_hbm, as_v)
    pltpu.sync_copy(ad_hbm, ad_v)

    zv = jnp.zeros((L,), jnp.float32)

    def zero_body(i, c):
        denom_v[pl.ds(i * L, L)] = zv
        return c
    lax.fori_loop(0, NP // L, zero_body, 0)

    neg = jnp.full((L,), -1e30, jnp.float32)

    def max_body(i, carry):
        ms, md = carry
        ms = jnp.maximum(ms, as_v[pl.ds(i * L, L)])
        md = jnp.maximum(md, ad_v[pl.ds(i * L, L)])
        return ms, md
    ms, md = lax.fori_loop(0, N // L, max_body, (neg, neg))

    lanes = lax.iota(jnp.int32, L)

    def allmax(v):
        for step in (8, 4, 2, 1):
            red_a[pl.ds(0, L)] = v
            v = jnp.maximum(v, plsc.load_gather(red_a, [lanes ^ step]))
        return v

    m16 = allmax(ms) + allmax(md)
    m_tot = jnp.where(m16 >= 0.0, m16, 0.2 * m16)

    def edge_body(j, c):
        for kk in range(KB // L):
            s16 = pkd2_v[j, pl.ds(kk * L, L)]
            d16 = pkd2_v[j, pl.ds(KB + kk * L, L)]
            a1 = plsc.load_gather(as_v, [s16])
            a2 = plsc.load_gather(ad_v, [d16])
            e = a1 + a2
            e = jnp.where(e >= 0.0, e, 0.2 * e)
            ee = jnp.exp(e - m_tot)
            pkd3_v[j, pl.ds(kk * L, L)] = s16
            pkd3_v[j, pl.ds(KB + kk * L, L)] = d16
            pkd3_v[j, pl.ds(2 * KB + kk * L, L)] = plsc.bitcast(
                ee, jnp.int32)
            plsc.addupdate_scatter(denom_v, [d16], ee)
        return c
    lax.fori_loop(0, NBLK, edge_body, 0)

    pltpu.sync_copy(pkd3_v, pkd3_hbm.at[w])

    pltpu.sync_copy(denom_v, dsh.at[sid])
    plsc.subcore_barrier()
    base = sid * DSL
    pltpu.sync_copy(dsh.at[0, pl.ds(base, DSL)], red_a)

    def red_body(r, c):
        pltpu.sync_copy(dsh.at[r, pl.ds(base, DSL)], red_b)
        for k in range(DSL // L):
            red_a[pl.ds(k * L, L)] = (red_a[pl.ds(k * L, L)]
                                      + red_b[pl.ds(k * L, L)])
        return c
    lax.fori_loop(1, NS, red_body, 0)
    pltpu.sync_copy(red_a, dpart_hbm.at[cid, pl.ds(base, DSL)])


_sc_edge = pl.kernel(
    _sc_edge_body,
    out_type=(jax.ShapeDtypeStruct((NW, NBLK, 3 * KB), jnp.int32),
              jax.ShapeDtypeStruct((NC, NP), jnp.float32)),
    mesh=_MESH,
    compiler_params=_SC_PARAMS,
    scratch_types=[
        pltpu.VMEM((NBLK, 2 * KB), jnp.int32),
        pltpu.VMEM((NBLK, 3 * KB), jnp.int32),
        pltpu.VMEM((N,), jnp.float32),
        pltpu.VMEM((N,), jnp.float32),
        pltpu.VMEM((NP,), jnp.float32),
        pltpu.VMEM((DSL,), jnp.float32),
        pltpu.VMEM((DSL,), jnp.float32),
        pltpu.VMEM_SHARED((NS, NP), jnp.float32),
    ],
)


def _sc_msg_body(h_hbm, pkd3_hbm, dpart_hbm, out_hbm,
                 pkd3_v, sdix_v, alphab_v, rec_v, d0_v, d1_v, rows_v,
                 acc_sh, rec_sh, gsem, ssem):
    cid = lax.axis_index("c")
    sid = lax.axis_index("s")
    w = cid * NS + sid

    zv = jnp.zeros((L,), jnp.float32)
    for j in range(ZR):
        for k in range(D // L):
            rows_v[0, j, pl.ds(k * L, L)] = zv
    base_rows = sid * ORT
    for r in range(ORT // ZR):
        pltpu.sync_copy(rows_v.at[0, pl.ds(0, ZR)],
                        acc_sh.at[pl.ds(base_rows + r * ZR, ZR)])

    @pl.when(sid == NS - 1)
    def _zero_tail():
        pltpu.sync_copy(rows_v.at[0, pl.ds(0, OREM)],
                        acc_sh.at[pl.ds(NS * ORT, OREM)])

    dbase = sid * DSL
    pltpu.sync_copy(dpart_hbm.at[0, pl.ds(dbase, DSL)], d0_v)
    pltpu.sync_copy(dpart_hbm.at[1, pl.ds(dbase, DSL)], d1_v)
    one = jnp.ones((L,), jnp.float32)
    for k in range(DSL // L):
        d0_v[pl.ds(k * L, L)] = one / (d0_v[pl.ds(k * L, L)]
                                       + d1_v[pl.ds(k * L, L)] + 1e-16)
    pltpu.sync_copy(d0_v, rec_sh.at[pl.ds(dbase, DSL)])

    plsc.subcore_barrier()
    pltpu.sync_copy(rec_sh, rec_v)

    def load_idx(j, s):
        pltpu.sync_copy(pkd3_hbm.at[w, j], pkd3_v.at[s])

    def start_gather(s):
        pltpu.async_copy(h_hbm.at[pkd3_v.at[s, pl.ds(0, KB)]],
                         rows_v.at[s], gsem.at[s])

    def start_scatter(s):
        pltpu.async_copy(rows_v.at[s], acc_sh.at[sdix_v.at[s]],
                         ssem.at[s], add=True)

    def wait_gather(s):
        pltpu.make_async_copy(h_hbm.at[pkd3_v.at[s, pl.ds(0, KB)]],
                              rows_v.at[s], gsem.at[s]).wait()

    def wait_scatter(s):
        pltpu.make_async_copy(rows_v.at[s], acc_sh.at[sdix_v.at[s]],
                              ssem.at[s]).wait()

    def compute(s):
        for k in range(KB // L):
            d16 = pkd3_v[s, pl.ds(KB + k * L, L)]
            sdix_v[s, pl.ds(k * L, L)] = d16
            ee = plsc.bitcast(pkd3_v[s, pl.ds(2 * KB + k * L, L)],
                              jnp.float32)
            alphab_v[s, pl.ds(k * L, L)] = (
                ee * plsc.load_gather(rec_v, [d16]))
        for k in range(KB // L):
            for rr in range(L):
                r = k * L + rr
                a16 = plsc.load_gather(
                    alphab_v, [jnp.full((L,), s, jnp.int32),
                               jnp.full((L,), r, jnp.int32)])
                for q in range(D // L):
                    rows_v[s, r, pl.ds(q * L, L)] = (
                        rows_v[s, r, pl.ds(q * L, L)] * a16)

    load_idx(0, 0)
    start_gather(0)

    load_idx(1, 1)
    start_gather(1)
    wait_gather(0)
    compute(0)
    start_scatter(0)

    def step(j, c):
        s = j % 2
        t = 1 - s
        load_idx(j + 1, t)
        wait_scatter(t)
        start_gather(t)
        wait_gather(s)
        compute(s)
        start_scatter(s)
        return c
    lax.fori_loop(1, NBLK - 1, step, 0)

    wait_scatter(1)
    wait_gather(0)
    compute(0)
    pltpu.sync_copy(rows_v.at[0], acc_sh.at[sdix_v.at[0]], add=True)

    plsc.subcore_barrier()
    pltpu.sync_copy(acc_sh.at[pl.ds(base_rows, ORT)],
                    out_hbm.at[cid, pl.ds(base_rows, ORT)])

    @pl.when(sid == NS - 1)
    def _copy_tail():
        pltpu.sync_copy(acc_sh.at[pl.ds(NS * ORT, OREM)],
                        out_hbm.at[cid, pl.ds(NS * ORT, OREM)])


_sc_msg = pl.kernel(
    _sc_msg_body,
    out_type=jax.ShapeDtypeStruct((NC, N, D), jnp.float32),
    mesh=_MESH,
    compiler_params=_SC_PARAMS,
    scratch_types=[
        pltpu.VMEM((2, 3 * KB), jnp.int32),
        pltpu.VMEM((2, KB), jnp.int32),
        pltpu.VMEM((2, KB), jnp.float32),
        pltpu.VMEM((NP,), jnp.float32),
        pltpu.VMEM((DSL,), jnp.float32),
        pltpu.VMEM((DSL,), jnp.float32),
        pltpu.VMEM((2, KB, D), jnp.float32),
        pltpu.VMEM_SHARED((N, D), jnp.float32),
        pltpu.VMEM_SHARED((NP,), jnp.float32),
        pltpu.SemaphoreType.DMA((2,)),
        pltpu.SemaphoreType.DMA((2,)),
    ],
)


def kernel(x, edge_index, W1, a_src1, a_dst1, b1, W2, a_src2, a_dst2, b2):
    srcb = edge_index[0].reshape(NW, NBLK, KB)
    dstb = edge_index[1].reshape(NW, NBLK, KB)
    pkd = jnp.concatenate([srcb, dstb], axis=2)

    h1, as1, ad1 = _tc_head1(x, W1, a_src1, a_dst1)
    pkd3_1, dpart1 = _sc_edge(pkd, as1.reshape(N), ad1.reshape(N))
    p1 = _sc_msg(h1, pkd3_1, dpart1)

    h2, as2, ad2 = _tc_head2(p1, b1, W2, a_src2, a_dst2)
    pkd3_2, dpart2 = _sc_edge(pkd, as2.reshape(N), ad2.reshape(N))
    p2 = _sc_msg(h2, pkd3_2, dpart2)

    return _tc_out(p2, b2)


# speedup vs baseline: 38.9009x; 1.0225x over previous
import jax
import jax.numpy as jnp
from jax import lax
from jax.experimental import pallas as pl
from jax.experimental.pallas import tpu as pltpu
from jax.experimental.pallas import tpu_sc as plsc

N = 10000
E = 320000
D = 128
NC = 2
NS = 16
NW = NC * NS
L = 16
EW = E // NW
KB = 80
NBLK = EW // KB
NP = 10240
DSL = NP // NS
G = 8
NBP = 128
NG = NBP // G
ORT = 624
OREM = N - NS * ORT
ZR = 24

_MESH = plsc.VectorSubcoreMesh(core_axis_name="c", subcore_axis_name="s")
_SC_PARAMS = pltpu.CompilerParams(needs_layout_passes=False)


def _tc_head1_body(x_ref, w_ref, asr_ref, adr_ref, h_ref, as_ref, ad_ref):
    h = jnp.dot(x_ref[...], w_ref[...], preferred_element_type=jnp.float32)
    h_ref[...] = h
    as_ref[...] = jnp.sum(h * asr_ref[...][None, :], axis=1, keepdims=True)
    ad_ref[...] = jnp.sum(h * adr_ref[...][None, :], axis=1, keepdims=True)


def _tc_head2_body(p_ref, b_ref, w_ref, asr_ref, adr_ref, h_ref, as_ref,
                   ad_ref):
    hid = jnp.maximum(p_ref[0] + p_ref[1] + b_ref[...][None, :], 0.0)
    h = jnp.dot(hid, w_ref[...], preferred_element_type=jnp.float32)
    h_ref[...] = h
    as_ref[...] = jnp.sum(h * asr_ref[...][None, :], axis=1, keepdims=True)
    ad_ref[...] = jnp.sum(h * adr_ref[...][None, :], axis=1, keepdims=True)


def _tc_out_body(p_ref, b_ref, o_ref):
    o = p_ref[0] + p_ref[1] + b_ref[...][None, :]
    m = jnp.max(o, axis=1, keepdims=True)
    ex = jnp.exp(o - m)
    o_ref[...] = o - m - jnp.log(jnp.sum(ex, axis=1, keepdims=True))


_BN = 2000

_head_out = [
    jax.ShapeDtypeStruct((N, D), jnp.float32),
    jax.ShapeDtypeStruct((N, 1), jnp.float32),
    jax.ShapeDtypeStruct((N, 1), jnp.float32),
]
_head_out_specs = [
    pl.BlockSpec((_BN, D), lambda i: (i, 0)),
    pl.BlockSpec((_BN, 1), lambda i: (i, 0)),
    pl.BlockSpec((_BN, 1), lambda i: (i, 0)),
]
_full_mat = pl.BlockSpec((D, D), lambda i: (0, 0))
_full_vec = pl.BlockSpec((D,), lambda i: (0,))


def _tc_head1(x, W, a_src, a_dst):
    return pl.pallas_call(
        _tc_head1_body,
        grid=(N // _BN,),
        in_specs=[pl.BlockSpec((_BN, D), lambda i: (i, 0)), _full_mat,
                  _full_vec, _full_vec],
        out_specs=_head_out_specs,
        out_shape=_head_out,
    )(x, W, a_src, a_dst)


def _tc_head2(p, b, W, a_src, a_dst):
    return pl.pallas_call(
        _tc_head2_body,
        grid=(N // _BN,),
        in_specs=[pl.BlockSpec((NC, _BN, D), lambda i: (0, i, 0)), _full_vec,
                  _full_mat, _full_vec, _full_vec],
        out_specs=_head_out_specs,
        out_shape=_head_out,
    )(p, b, W, a_src, a_dst)


def _tc_out(p, b):
    return pl.pallas_call(
        _tc_out_body,
        grid=(N // _BN,),
        in_specs=[pl.BlockSpec((NC, _BN, D), lambda i: (0, i, 0)), _full_vec],
        out_specs=pl.BlockSpec((_BN, D), lambda i: (i, 0)),
        out_shape=jax.ShapeDtypeStruct((N, D), jnp.float32),
    )(p, b)


def _sc_edge_body(pkd2_hbm, as_hbm, ad_hbm, pkd3_hbm, dpart_hbm,
                  pkd2_v, pkd3_v, as_v, ad_v, denom_v, red_a, red_b,
                  dsh):
    cid = lax.axis_index("c")
    sid = lax.axis_index("s")
    w = cid * NS + sid

    pltpu.sync_copy(pkd2_hbm.at[w], pkd2_v)
    pltpu.sync_copy(as_hbm, as_v)
    pltpu.sync_copy(ad_hbm, ad_v)

    zv = jnp.zeros((L,), jnp.float32)

    def zero_body(i, c):
        denom_v[pl.ds(i * L, L)] = zv
        return c
    lax.fori_loop(0, NP // L, zero_body, 0)

    neg = jnp.full((L,), -1e30, jnp.float32)

    def max_body(i, carry):
        ms, md = carry
        ms = jnp.maximum(ms, as_v[pl.ds(i * L, L)])
        md = jnp.maximum(md, ad_v[pl.ds(i * L, L)])
        return ms, md
    ms, md = lax.fori_loop(0, N // L, max_body, (neg, neg))

    lanes = lax.iota(jnp.int32, L)

    def allmax(v):
        for step in (8, 4, 2, 1):
            red_a[pl.ds(0, L)] = v
            v = jnp.maximum(v, plsc.load_gather(red_a, [lanes ^ step]))
        return v

    m16 = allmax(ms) + allmax(md)
    m_tot = jnp.where(m16 >= 0.0, m16, 0.2 * m16)

    def edge_body(j, c):
        for kk in range(KB // L):
            s16 = pkd2_v[j, pl.ds(kk * L, L)]
            d16 = pkd2_v[j, pl.ds(KB + kk * L, L)]
            a1 = plsc.load_gather(as_v, [s16])
            a2 = plsc.load_gather(ad_v, [d16])
            e = a1 + a2
            e = jnp.where(e >= 0.0, e, 0.2 * e)
            ee = jnp.exp(e - m_tot)
            pkd3_v[j, pl.ds(kk * L, L)] = s16
            pkd3_v[j, pl.ds(KB + kk * L, L)] = d16
            pkd3_v[j, pl.ds(2 * KB + kk * L, L)] = plsc.bitcast(
                ee, jnp.int32)
            plsc.addupdate_scatter(denom_v, [d16], ee)
        return c
    lax.fori_loop(0, NBLK, edge_body, 0)

    pltpu.sync_copy(pkd3_v, pkd3_hbm.at[w])

    pltpu.sync_copy(denom_v, dsh.at[sid])
    plsc.subcore_barrier()
    base = sid * DSL
    pltpu.sync_copy(dsh.at[0, pl.ds(base, DSL)], red_a)

    def red_body(r, c):
        pltpu.sync_copy(dsh.at[r, pl.ds(base, DSL)], red_b)
        for k in range(DSL // L):
            red_a[pl.ds(k * L, L)] = (red_a[pl.ds(k * L, L)]
                                      + red_b[pl.ds(k * L, L)])
        return c
    lax.fori_loop(1, NS, red_body, 0)
    pltpu.sync_copy(red_a, dpart_hbm.at[cid, pl.ds(base, DSL)])


_sc_edge = pl.kernel(
    _sc_edge_body,
    out_type=(jax.ShapeDtypeStruct((NW, NBP, 3 * KB), jnp.int32),
              jax.ShapeDtypeStruct((NC, NP), jnp.float32)),
    mesh=_MESH,
    compiler_params=_SC_PARAMS,
    scratch_types=[
        pltpu.VMEM((NBLK, 2 * KB), jnp.int32),
        pltpu.VMEM((NBP, 3 * KB), jnp.int32),
        pltpu.VMEM((N,), jnp.float32),
        pltpu.VMEM((N,), jnp.float32),
        pltpu.VMEM((NP,), jnp.float32),
        pltpu.VMEM((DSL,), jnp.float32),
        pltpu.VMEM((DSL,), jnp.float32),
        pltpu.VMEM_SHARED((NS, NP), jnp.float32),
    ],
)


def _sc_msg_body(h_hbm, pkd3_hbm, dpart_hbm, out_hbm,
                 pkd3g_v, sdix_v, alphab_v, rec_v, d0_v, d1_v, rows_v,
                 acc_sh, rec_sh, gsem, ssem):
    cid = lax.axis_index("c")
    sid = lax.axis_index("s")
    w = cid * NS + sid

    zv = jnp.zeros((L,), jnp.float32)
    for j in range(ZR):
        for k in range(D // L):
            rows_v[0, j, pl.ds(k * L, L)] = zv
    base_rows = sid * ORT
    for r in range(ORT // ZR):
        pltpu.sync_copy(rows_v.at[0, pl.ds(0, ZR)],
                        acc_sh.at[pl.ds(base_rows + r * ZR, ZR)])

    @pl.when(sid == NS - 1)
    def _zero_tail():
        pltpu.sync_copy(rows_v.at[0, pl.ds(0, OREM)],
                        acc_sh.at[pl.ds(NS * ORT, OREM)])

    dbase = sid * DSL
    pltpu.sync_copy(dpart_hbm.at[0, pl.ds(dbase, DSL)], d0_v)
    pltpu.sync_copy(dpart_hbm.at[1, pl.ds(dbase, DSL)], d1_v)
    one = jnp.ones((L,), jnp.float32)
    for k in range(DSL // L):
        d0_v[pl.ds(k * L, L)] = one / (d0_v[pl.ds(k * L, L)]
                                       + d1_v[pl.ds(k * L, L)] + 1e-16)
    pltpu.sync_copy(d0_v, rec_sh.at[pl.ds(dbase, DSL)])

    plsc.subcore_barrier()
    pltpu.sync_copy(rec_sh, rec_v)

    def load_grp(g, gs):
        pltpu.sync_copy(pkd3_hbm.at[w, pl.ds(g * G, G)], pkd3g_v.at[gs])

    def start_gather(s, j):
        pltpu.async_copy(
            h_hbm.at[pkd3g_v.at[(j // G) % 2, j % G, pl.ds(0, KB)]],
            rows_v.at[s], gsem.at[s])

    def start_scatter(s):
        pltpu.async_copy(rows_v.at[s], acc_sh.at[sdix_v.at[s]],
                         ssem.at[s], add=True)

    def wait_gather(s, j):
        pltpu.make_async_copy(
            h_hbm.at[pkd3g_v.at[(j // G) % 2, j % G, pl.ds(0, KB)]],
            rows_v.at[s], gsem.at[s]).wait()

    def wait_scatter(s):
        pltpu.make_async_copy(rows_v.at[s], acc_sh.at[sdix_v.at[s]],
                              ssem.at[s]).wait()

    def compute(s, j):
        gg = (j // G) % 2
        bb = j % G
        for k in range(KB // L):
            d16 = pkd3g_v[gg, bb, pl.ds(KB + k * L, L)]
            sdix_v[s, pl.ds(k * L, L)] = d16
            ee = plsc.bitcast(pkd3g_v[gg, bb, pl.ds(2 * KB + k * L, L)],
                              jnp.float32)
            alphab_v[s, pl.ds(k * L, L)] = (
                ee * plsc.load_gather(rec_v, [d16]))
        for k in range(KB // L):
            for rr in range(L):
                r = k * L + rr
                a16 = plsc.load_gather(
                    alphab_v, [jnp.full((L,), s, jnp.int32),
                               jnp.full((L,), r, jnp.int32)])
                for q in range(D // L):
                    rows_v[s, r, pl.ds(q * L, L)] = (
                        rows_v[s, r, pl.ds(q * L, L)] * a16)

    load_grp(0, 0)
    start_gather(0, 0)

    load_grp(1, 1)
    start_gather(1, 1)
    wait_gather(0, 0)
    compute(0, 0)
    start_scatter(0)

    def step(j, c):
        s = j % 2
        t = 1 - s

        @pl.when((j % G == 0) & (j < (NG - 1) * G))
        def _load_next_group():
            g1 = j // G + 1
            load_grp(g1, g1 % 2)

        wait_scatter(t)
        start_gather(t, j + 1)
        wait_gather(s, j)
        compute(s, j)
        start_scatter(s)
        return c
    lax.fori_loop(1, NBLK - 1, step, 0)

    wait_scatter(1)
    wait_gather(0, NBLK - 1)
    compute(0, NBLK - 1)
    pltpu.sync_copy(rows_v.at[0], acc_sh.at[sdix_v.at[0]], add=True)

    plsc.subcore_barrier()
    pltpu.sync_copy(acc_sh.at[pl.ds(base_rows, ORT)],
                    out_hbm.at[cid, pl.ds(base_rows, ORT)])

    @pl.when(sid == NS - 1)
    def _copy_tail():
        pltpu.sync_copy(acc_sh.at[pl.ds(NS * ORT, OREM)],
                        out_hbm.at[cid, pl.ds(NS * ORT, OREM)])


_sc_msg = pl.kernel(
    _sc_msg_body,
    out_type=jax.ShapeDtypeStruct((NC, N, D), jnp.float32),
    mesh=_MESH,
    compiler_params=_SC_PARAMS,
    scratch_types=[
        pltpu.VMEM((2, G, 3 * KB), jnp.int32),
        pltpu.VMEM((2, KB), jnp.int32),
        pltpu.VMEM((2, KB), jnp.float32),
        pltpu.VMEM((NP,), jnp.float32),
        pltpu.VMEM((DSL,), jnp.float32),
        pltpu.VMEM((DSL,), jnp.float32),
        pltpu.VMEM((2, KB, D), jnp.float32),
        pltpu.VMEM_SHARED((N, D), jnp.float32),
        pltpu.VMEM_SHARED((NP,), jnp.float32),
        pltpu.SemaphoreType.DMA((2,)),
        pltpu.SemaphoreType.DMA((2,)),
    ],
)


def kernel(x, edge_index, W1, a_src1, a_dst1, b1, W2, a_src2, a_dst2, b2):
    srcb = edge_index[0].reshape(NW, NBLK, KB)
    dstb = edge_index[1].reshape(NW, NBLK, KB)
    pkd = jnp.concatenate([srcb, dstb], axis=2)

    h1, as1, ad1 = _tc_head1(x, W1, a_src1, a_dst1)
    pkd3_1, dpart1 = _sc_edge(pkd, as1.reshape(N), ad1.reshape(N))
    p1 = _sc_msg(h1, pkd3_1, dpart1)

    h2, as2, ad2 = _tc_head2(p1, b1, W2, a_src2, a_dst2)
    pkd3_2, dpart2 = _sc_edge(pkd, as2.reshape(N), ad2.reshape(N))
    p2 = _sc_msg(h2, pkd3_2, dpart2)

    return _tc_out(p2, b2)


# speedup vs baseline: 39.4735x; 1.0147x over previous
import jax
import jax.numpy as jnp
from jax import lax
from jax.experimental import pallas as pl
from jax.experimental.pallas import tpu as pltpu
from jax.experimental.pallas import tpu_sc as plsc

N = 10000
E = 320000
D = 128
NC = 2
NS = 16
NW = NC * NS
L = 16
EW = E // NW
KB = 80
NBLK = EW // KB
NP = 10240
DSL = NP // NS
G = 8
NBP = 128
NG = NBP // G
ORT = 624
OREM = N - NS * ORT
ZR = 24

_MESH = plsc.VectorSubcoreMesh(core_axis_name="c", subcore_axis_name="s")
_SC_PARAMS = pltpu.CompilerParams(needs_layout_passes=False)


def _tc_head1_body(x_ref, w_ref, asr_ref, adr_ref, h_ref, as_ref, ad_ref):
    h = jnp.dot(x_ref[...], w_ref[...], preferred_element_type=jnp.float32)
    h_ref[...] = h
    as_ref[...] = jnp.sum(h * asr_ref[...][None, :], axis=1, keepdims=True)
    ad_ref[...] = jnp.sum(h * adr_ref[...][None, :], axis=1, keepdims=True)


def _tc_head2_body(p_ref, b_ref, w_ref, asr_ref, adr_ref, h_ref, as_ref,
                   ad_ref):
    hid = jnp.maximum(p_ref[0] + p_ref[1] + b_ref[...][None, :], 0.0)
    h = jnp.dot(hid, w_ref[...], preferred_element_type=jnp.float32)
    h_ref[...] = h
    as_ref[...] = jnp.sum(h * asr_ref[...][None, :], axis=1, keepdims=True)
    ad_ref[...] = jnp.sum(h * adr_ref[...][None, :], axis=1, keepdims=True)


def _tc_out_body(p_ref, b_ref, o_ref):
    o = p_ref[0] + p_ref[1] + b_ref[...][None, :]
    m = jnp.max(o, axis=1, keepdims=True)
    ex = jnp.exp(o - m)
    o_ref[...] = o - m - jnp.log(jnp.sum(ex, axis=1, keepdims=True))


_BN = 2000

_head_out = [
    jax.ShapeDtypeStruct((N, D), jnp.float32),
    jax.ShapeDtypeStruct((N, 1), jnp.float32),
    jax.ShapeDtypeStruct((N, 1), jnp.float32),
]
_head_out_specs = [
    pl.BlockSpec((_BN, D), lambda i: (i, 0)),
    pl.BlockSpec((_BN, 1), lambda i: (i, 0)),
    pl.BlockSpec((_BN, 1), lambda i: (i, 0)),
]
_full_mat = pl.BlockSpec((D, D), lambda i: (0, 0))
_full_vec = pl.BlockSpec((D,), lambda i: (0,))


def _tc_head1(x, W, a_src, a_dst):
    return pl.pallas_call(
        _tc_head1_body,
        grid=(N // _BN,),
        in_specs=[pl.BlockSpec((_BN, D), lambda i: (i, 0)), _full_mat,
                  _full_vec, _full_vec],
        out_specs=_head_out_specs,
        out_shape=_head_out,
    )(x, W, a_src, a_dst)


def _tc_head2(p, b, W, a_src, a_dst):
    return pl.pallas_call(
        _tc_head2_body,
        grid=(N // _BN,),
        in_specs=[pl.BlockSpec((NC, _BN, D), lambda i: (0, i, 0)), _full_vec,
                  _full_mat, _full_vec, _full_vec],
        out_specs=_head_out_specs,
        out_shape=_head_out,
    )(p, b, W, a_src, a_dst)


def _tc_out(p, b):
    return pl.pallas_call(
        _tc_out_body,
        grid=(N // _BN,),
        in_specs=[pl.BlockSpec((NC, _BN, D), lambda i: (0, i, 0)), _full_vec],
        out_specs=pl.BlockSpec((_BN, D), lambda i: (i, 0)),
        out_shape=jax.ShapeDtypeStruct((N, D), jnp.float32),
    )(p, b)


def _sc_edge_body(pkd2_hbm, as_hbm, ad_hbm, pkd3_hbm, dpart_hbm,
                  pkd2_v, pkd3_v, as_v, ad_v, denom_v, red_a, red_b,
                  dsh, lsem):
    cid = lax.axis_index("c")
    sid = lax.axis_index("s")
    w = cid * NS + sid

    c_pkd2 = pltpu.async_copy(pkd2_hbm.at[w], pkd2_v, lsem)
    c_as = pltpu.async_copy(as_hbm, as_v, lsem)
    c_ad = pltpu.async_copy(ad_hbm, ad_v, lsem)

    zv = jnp.zeros((L,), jnp.float32)

    def zero_body(i, c):
        denom_v[pl.ds(i * L, L)] = zv
        return c
    lax.fori_loop(0, NP // L, zero_body, 0)

    c_pkd2.wait()
    c_as.wait()
    c_ad.wait()

    neg = jnp.full((L,), -1e30, jnp.float32)

    def max_body(i, carry):
        ms, md = carry
        ms = jnp.maximum(ms, as_v[pl.ds(i * L, L)])
        md = jnp.maximum(md, ad_v[pl.ds(i * L, L)])
        return ms, md
    ms, md = lax.fori_loop(0, N // L, max_body, (neg, neg))

    lanes = lax.iota(jnp.int32, L)

    def allmax(v):
        for step in (8, 4, 2, 1):
            red_a[pl.ds(0, L)] = v
            v = jnp.maximum(v, plsc.load_gather(red_a, [lanes ^ step]))
        return v

    m16 = allmax(ms) + allmax(md)
    m_tot = jnp.where(m16 >= 0.0, m16, 0.2 * m16)

    def edge_body(j, c):
        for kk in range(KB // L):
            s16 = pkd2_v[j, pl.ds(kk * L, L)]
            d16 = pkd2_v[j, pl.ds(KB + kk * L, L)]
            a1 = plsc.load_gather(as_v, [s16])
            a2 = plsc.load_gather(ad_v, [d16])
            e = a1 + a2
            e = jnp.where(e >= 0.0, e, 0.2 * e)
            ee = jnp.exp(e - m_tot)
            pkd3_v[j, pl.ds(kk * L, L)] = s16
            pkd3_v[j, pl.ds(KB + kk * L, L)] = d16
            pkd3_v[j, pl.ds(2 * KB + kk * L, L)] = plsc.bitcast(
                ee, jnp.int32)
            plsc.addupdate_scatter(denom_v, [d16], ee)
        return c
    lax.fori_loop(0, NBLK, edge_body, 0)

    c_pkd3 = pltpu.async_copy(pkd3_v, pkd3_hbm.at[w], lsem)

    pltpu.sync_copy(denom_v, dsh.at[sid])
    plsc.subcore_barrier()
    base = sid * DSL
    pltpu.sync_copy(dsh.at[0, pl.ds(base, DSL)], red_a)

    def red_body(r, c):
        pltpu.sync_copy(dsh.at[r, pl.ds(base, DSL)], red_b)
        for k in range(DSL // L):
            red_a[pl.ds(k * L, L)] = (red_a[pl.ds(k * L, L)]
                                      + red_b[pl.ds(k * L, L)])
        return c
    lax.fori_loop(1, NS, red_body, 0)
    pltpu.sync_copy(red_a, dpart_hbm.at[cid, pl.ds(base, DSL)])
    c_pkd3.wait()


_sc_edge = pl.kernel(
    _sc_edge_body,
    out_type=(jax.ShapeDtypeStruct((NW, NBP, 3 * KB), jnp.int32),
              jax.ShapeDtypeStruct((NC, NP), jnp.float32)),
    mesh=_MESH,
    compiler_params=_SC_PARAMS,
    scratch_types=[
        pltpu.VMEM((NBLK, 2 * KB), jnp.int32),
        pltpu.VMEM((NBP, 3 * KB), jnp.int32),
        pltpu.VMEM((N,), jnp.float32),
        pltpu.VMEM((N,), jnp.float32),
        pltpu.VMEM((NP,), jnp.float32),
        pltpu.VMEM((DSL,), jnp.float32),
        pltpu.VMEM((DSL,), jnp.float32),
        pltpu.VMEM_SHARED((NS, NP), jnp.float32),
        pltpu.SemaphoreType.DMA,
    ],
)


def _sc_msg_body(h_hbm, pkd3_hbm, dpart_hbm, out_hbm,
                 pkd3g_v, sdix_v, alphab_v, rec_v, d0_v, d1_v, rows_v,
                 acc_sh, rec_sh, gsem, ssem):
    cid = lax.axis_index("c")
    sid = lax.axis_index("s")
    w = cid * NS + sid

    zv = jnp.zeros((L,), jnp.float32)
    for j in range(ZR):
        for k in range(D // L):
            rows_v[0, j, pl.ds(k * L, L)] = zv
    base_rows = sid * ORT
    for r in range(ORT // ZR):
        pltpu.sync_copy(rows_v.at[0, pl.ds(0, ZR)],
                        acc_sh.at[pl.ds(base_rows + r * ZR, ZR)])

    @pl.when(sid == NS - 1)
    def _zero_tail():
        pltpu.sync_copy(rows_v.at[0, pl.ds(0, OREM)],
                        acc_sh.at[pl.ds(NS * ORT, OREM)])

    dbase = sid * DSL
    pltpu.sync_copy(dpart_hbm.at[0, pl.ds(dbase, DSL)], d0_v)
    pltpu.sync_copy(dpart_hbm.at[1, pl.ds(dbase, DSL)], d1_v)
    one = jnp.ones((L,), jnp.float32)
    for k in range(DSL // L):
        d0_v[pl.ds(k * L, L)] = one / (d0_v[pl.ds(k * L, L)]
                                       + d1_v[pl.ds(k * L, L)] + 1e-16)
    pltpu.sync_copy(d0_v, rec_sh.at[pl.ds(dbase, DSL)])

    plsc.subcore_barrier()
    pltpu.sync_copy(rec_sh, rec_v)

    def load_grp(g, gs):
        pltpu.sync_copy(pkd3_hbm.at[w, pl.ds(g * G, G)], pkd3g_v.at[gs])

    def start_gather(s, j):
        pltpu.async_copy(
            h_hbm.at[pkd3g_v.at[(j // G) % 2, j % G, pl.ds(0, KB)]],
            rows_v.at[s], gsem.at[s])

    def start_scatter(s):
        pltpu.async_copy(rows_v.at[s], acc_sh.at[sdix_v.at[s]],
                         ssem.at[s], add=True)

    def wait_gather(s, j):
        pltpu.make_async_copy(
            h_hbm.at[pkd3g_v.at[(j // G) % 2, j % G, pl.ds(0, KB)]],
            rows_v.at[s], gsem.at[s]).wait()

    def wait_scatter(s):
        pltpu.make_async_copy(rows_v.at[s], acc_sh.at[sdix_v.at[s]],
                              ssem.at[s]).wait()

    def compute(s, j):
        gg = (j // G) % 2
        bb = j % G
        for k in range(KB // L):
            d16 = pkd3g_v[gg, bb, pl.ds(KB + k * L, L)]
            sdix_v[s, pl.ds(k * L, L)] = d16
            ee = plsc.bitcast(pkd3g_v[gg, bb, pl.ds(2 * KB + k * L, L)],
                              jnp.float32)
            alphab_v[s, pl.ds(k * L, L)] = (
                ee * plsc.load_gather(rec_v, [d16]))
        for k in range(KB // L):
            for rr in range(L):
                r = k * L + rr
                a16 = plsc.load_gather(
                    alphab_v, [jnp.full((L,), s, jnp.int32),
                               jnp.full((L,), r, jnp.int32)])
                for q in range(D // L):
                    rows_v[s, r, pl.ds(q * L, L)] = (
                        rows_v[s, r, pl.ds(q * L, L)] * a16)

    load_grp(0, 0)
    start_gather(0, 0)

    load_grp(1, 1)
    start_gather(1, 1)
    wait_gather(0, 0)
    compute(0, 0)
    start_scatter(0)

    def step(j, c):
        s = j % 2
        t = 1 - s

        @pl.when((j % G == 0) & (j < (NG - 1) * G))
        def _load_next_group():
            g1 = j // G + 1
            load_grp(g1, g1 % 2)

        wait_scatter(t)
        start_gather(t, j + 1)
        wait_gather(s, j)
        compute(s, j)
        start_scatter(s)
        return c
    lax.fori_loop(1, NBLK - 1, step, 0)

    wait_scatter(1)
    wait_gather(0, NBLK - 1)
    compute(0, NBLK - 1)
    pltpu.sync_copy(rows_v.at[0], acc_sh.at[sdix_v.at[0]], add=True)

    plsc.subcore_barrier()
    pltpu.sync_copy(acc_sh.at[pl.ds(base_rows, ORT)],
                    out_hbm.at[cid, pl.ds(base_rows, ORT)])

    @pl.when(sid == NS - 1)
    def _copy_tail():
        pltpu.sync_copy(acc_sh.at[pl.ds(NS * ORT, OREM)],
                        out_hbm.at[cid, pl.ds(NS * ORT, OREM)])


_sc_msg = pl.kernel(
    _sc_msg_body,
    out_type=jax.ShapeDtypeStruct((NC, N, D), jnp.float32),
    mesh=_MESH,
    compiler_params=_SC_PARAMS,
    scratch_types=[
        pltpu.VMEM((2, G, 3 * KB), jnp.int32),
        pltpu.VMEM((2, KB), jnp.int32),
        pltpu.VMEM((2, KB), jnp.float32),
        pltpu.VMEM((NP,), jnp.float32),
        pltpu.VMEM((DSL,), jnp.float32),
        pltpu.VMEM((DSL,), jnp.float32),
        pltpu.VMEM((2, KB, D), jnp.float32),
        pltpu.VMEM_SHARED((N, D), jnp.float32),
        pltpu.VMEM_SHARED((NP,), jnp.float32),
        pltpu.SemaphoreType.DMA((2,)),
        pltpu.SemaphoreType.DMA((2,)),
    ],
)


def kernel(x, edge_index, W1, a_src1, a_dst1, b1, W2, a_src2, a_dst2, b2):
    srcb = edge_index[0].reshape(NW, NBLK, KB)
    dstb = edge_index[1].reshape(NW, NBLK, KB)
    pkd = jnp.concatenate([srcb, dstb], axis=2)

    h1, as1, ad1 = _tc_head1(x, W1, a_src1, a_dst1)
    pkd3_1, dpart1 = _sc_edge(pkd, as1.reshape(N), ad1.reshape(N))
    p1 = _sc_msg(h1, pkd3_1, dpart1)

    h2, as2, ad2 = _tc_head2(p1, b1, W2, a_src2, a_dst2)
    pkd3_2, dpart2 = _sc_edge(pkd, as2.reshape(N), ad2.reshape(N))
    p2 = _sc_msg(h2, pkd3_2, dpart2)

    return _tc_out(p2, b2)
